# Initial kernel scaffold; baseline (speedup 1.0000x reference)
import jax
import jax.numpy as jnp
from jax.experimental import pallas as pl


def kernel(s, v, edge_index, r_ij, W1, b1, W2, b2, W3, b3, W4, b4, W5, b5):
    raise NotImplementedError("write your pallas kernel here")


# speedup vs baseline: 11.1910x; 11.1910x over previous
import functools

import jax
import jax.numpy as jnp
from jax import lax
from jax.experimental import pallas as pl
from jax.experimental.pallas import tpu as pltpu
from jax.experimental.pallas import tpu_sc as plsc

CUTOFF = 5.0
NC = 2
NP = 2
NS = 16
LANES = 16


def _node_tables_body(s_ref, v_ref, w4_ref, b4_ref, w5_ref, b5_ref, r3_ref, a_ref):
    F = s_ref.shape[1]
    HV = 3 * (F // (NC * NP))
    HS = F // (NC * NP)
    sb = s_ref[...]
    phi_s = jax.nn.silu(
        jnp.dot(sb, w4_ref[...], preferred_element_type=jnp.float32) + b4_ref[...])
    phi2 = jax.nn.silu(
        jnp.dot(sb, w5_ref[...], preferred_element_type=jnp.float32) + b5_ref[...])
    r3 = r3_ref[...]
    pvv3 = jnp.dot(phi2[:, :F], r3, preferred_element_type=jnp.float32)
    pvs3 = jnp.dot(phi2[:, F:], r3, preferred_element_type=jnp.float32)
    u_vv = v_ref[...] * pvv3
    for c in range(NC):
        for p in range(NP):
            q = c * NP + p
            a_ref[c, p] = jnp.concatenate(
                [u_vv[:, q * HV:(q + 1) * HV],
                 pvs3[:, q * HV:(q + 1) * HV],
                 phi_s[:, q * HS:(q + 1) * HS]], axis=1)


def _node_tables(s, v384, W4, b4, W5p, b5p, R3, bn=400):
    N, F = s.shape
    W = 7 * (F // (NC * NP))
    grid = (N // bn,)
    return pl.pallas_call(
        _node_tables_body,
        grid=grid,
        in_specs=[
            pl.BlockSpec((bn, F), lambda i: (i, 0)),
            pl.BlockSpec((bn, 3 * F), lambda i: (i, 0)),
            pl.BlockSpec((F, F), lambda i: (0, 0)),
            pl.BlockSpec((1, F), lambda i: (0, 0)),
            pl.BlockSpec((F, 2 * F), lambda i: (0, 0)),
            pl.BlockSpec((1, 2 * F), lambda i: (0, 0)),
            pl.BlockSpec((F, 3 * F), lambda i: (0, 0)),
        ],
        out_specs=pl.BlockSpec((NC, NP, bn, W), lambda i: (0, 0, i, 0)),
        out_shape=jax.ShapeDtypeStruct((NC, NP, N, W), jnp.float32),
    )(s, v384, W4, b4, W5p, b5p, R3)


def _edge_tables_body(rt_ref, wcat_ref, bcat_ref, t3_ref, one_ref, b_ref, *, num_rbf, F):
    rt = rt_ref[...]
    be = rt.shape[1]
    r2 = jnp.sum(rt * rt, axis=0, keepdims=True)
    r = jnp.sqrt(r2)
    centers = lax.broadcasted_iota(jnp.int32, (num_rbf, be), 0).astype(
        jnp.float32) * (CUTOFF / (num_rbf - 1))
    inv_denom = 1.0 / (CUTOFF / num_rbf) ** 2
    rbf = jnp.exp(-0.5 * (r - centers) ** 2 * inv_denom)
    cut = 0.5 * (jnp.cos(jnp.pi * r / CUTOFF) + 1.0) * (r < CUTOFF)
    rbfc = rbf * cut
    bfull = lax.dot_general(
        rbfc, wcat_ref[...], (((0,), (0,)), ((), ())),
        preferred_element_type=jnp.float32) + bcat_ref[...]
    inv = 1.0 / (r + 1e-8)
    rnt = rt * inv
    rtile = lax.dot_general(
        rnt, t3_ref[...], (((0,), (0,)), ((), ())),
        preferred_element_type=jnp.float32)
    cutcol = lax.dot_general(
        cut, one_ref[...], (((0,), (0,)), ((), ())),
        preferred_element_type=jnp.float32)
    HV = 3 * (F // (NC * NP))
    HS = F // (NC * NP)
    wvv = bfull[:, :3 * F]
    wvs_r = bfull[:, 3 * F:6 * F] * rtile
    ws_c = bfull[:, 6 * F:] * cutcol
    for c in range(NC):
        for p in range(NP):
            q = c * NP + p
            b_ref[c, p] = jnp.concatenate(
                [wvv[:, q * HV:(q + 1) * HV],
                 wvs_r[:, q * HV:(q + 1) * HV],
                 ws_c[:, q * HS:(q + 1) * HS]], axis=1)


def _edge_tables(r_ij_T, Wcat, bcat, T3, one11, F, be=1280):
    num_rbf, WC = Wcat.shape
    E = r_ij_T.shape[1]
    W = 7 * (F // (NC * NP))
    grid = (E // be,)
    return pl.pallas_call(
        functools.partial(_edge_tables_body, num_rbf=num_rbf, F=F),
        grid=grid,
        in_specs=[
            pl.BlockSpec((3, be), lambda i: (0, i)),
            pl.BlockSpec((num_rbf, WC), lambda i: (0, 0)),
            pl.BlockSpec((1, WC), lambda i: (0, 0)),
            pl.BlockSpec((3, 3 * F), lambda i: (0, 0)),
            pl.BlockSpec((1, 1), lambda i: (0, 0)),
        ],
        out_specs=pl.BlockSpec((NC, NP, be, W), lambda i: (0, 0, i, 0)),
        out_shape=jax.ShapeDtypeStruct((NC, NP, E, W), jnp.float32),
    )(r_ij_T, Wcat, bcat, T3, one11)


def _sc_message_body(N, E, HV, W, C, a_hbm, b_hbm, src_hbm, dst_hbm,
                     dv_out, ds_out, srcv, gidx, dstv, arows, brows, msg, msgs,
                     shacc, ds_sh, sem_a, sem_b):
    c = lax.axis_index("c")
    sid = lax.axis_index("s")
    rpt = -(-(N // NS) // 8) * 8
    last = N - (NS - 1) * rpt
    base_row = sid * rpt
    n8 = jnp.where(sid == NS - 1, last // 8, rpt // 8)

    zero16 = jnp.zeros((LANES,), jnp.float32)

    def _zb(i, carry):
        def _zbi(j, carry2):
            msg[i, pl.ds(j * LANES, LANES)] = zero16
            return carry2
        msgs[i, :] = zero16
        return lax.fori_loop(0, HV // LANES, _zbi, carry)
    lax.fori_loop(0, C, _zb, 0)

    def _zds(i, carry):
        pltpu.sync_copy(msgs.at[pl.ds(0, 8)], ds_sh.at[pl.ds(base_row + i * 8, 8)])
        return carry
    lax.fori_loop(0, n8, _zds, 0)

    edges_per_tec = E // NS
    nchunks = edges_per_tec // C
    base_e = sid * edges_per_tec

    for p in range(NP):
        for i in range(8):
            for j in range(HV // LANES):
                msg[i, pl.ds(j * LANES, LANES)] = zero16

        def _zs(i, carry):
            pltpu.sync_copy(msg.at[pl.ds(0, 8)],
                            shacc.at[pl.ds(base_row + i * 8, 8)])
            return carry
        lax.fori_loop(0, n8, _zs, 0)
        plsc.subcore_barrier()

        def _chunk(g, carry):
            e0 = base_e + g * C
            pltpu.sync_copy(src_hbm.at[pl.ds(e0, C)], srcv)
            pltpu.sync_copy(dst_hbm.at[pl.ds(e0, C)], dstv)
            off = (c * NP + p) * N

            def _gi(k, carry2):
                sl = pl.ds(k * LANES, LANES)
                gidx[sl] = srcv[sl] + off
                return carry2
            lax.fori_loop(0, C // LANES, _gi, 0)

            cp_a = pltpu.async_copy(a_hbm.at[gidx], arows, sem_a)
            cp_b = pltpu.async_copy(
                b_hbm.at[pl.ds((c * NP + p) * E + e0, C)], brows, sem_b)
            cp_a.wait()
            cp_b.wait()

            def _edge(i, carry2):
                for j in range(HV // LANES):
                    sl = pl.ds(j * LANES, LANES)
                    s2 = pl.ds(HV + j * LANES, LANES)
                    msg[i, sl] = (arows[i, sl] * brows[i, sl]
                                  + arows[i, s2] * brows[i, s2])
                pp = jnp.zeros((LANES,), jnp.float32)
                for j in range((W - 2 * HV) // LANES):
                    sl = pl.ds(2 * HV + j * LANES, LANES)
                    pp = pp + arows[i, sl] * brows[i, sl]
                msgs[i, :] = pp
                return carry2
            lax.fori_loop(0, C, _edge, 0)

            pltpu.sync_copy(msg, shacc.at[dstv], add=True)
            pltpu.sync_copy(msgs, ds_sh.at[dstv], add=True)
            return carry

        lax.fori_loop(0, nchunks, _chunk, 0)
        plsc.subcore_barrier()

        @pl.when(sid != NS - 1)
        def _():
            pltpu.sync_copy(shacc.at[pl.ds(base_row, rpt)],
                            dv_out.at[c, p, pl.ds(base_row, rpt)])

        @pl.when(sid == NS - 1)
        def _():
            pltpu.sync_copy(shacc.at[pl.ds((NS - 1) * rpt, last)],
                            dv_out.at[c, p, pl.ds((NS - 1) * rpt, last)])

    @pl.when(sid != NS - 1)
    def _():
        pltpu.sync_copy(ds_sh.at[pl.ds(base_row, rpt)],
                        ds_out.at[c, pl.ds(base_row, rpt)])

    @pl.when(sid == NS - 1)
    def _():
        pltpu.sync_copy(ds_sh.at[pl.ds((NS - 1) * rpt, last)],
                        ds_out.at[c, pl.ds((NS - 1) * rpt, last)])


def _sc_message(A2, B2, src, dst, N, E, HV, W, C=80):
    mesh = plsc.VectorSubcoreMesh(
        core_axis_name="c", subcore_axis_name="s", num_cores=NC, num_subcores=NS)
    f = pl.kernel(
        functools.partial(_sc_message_body, N, E, HV, W, C),
        out_type=[
            jax.ShapeDtypeStruct((NC, NP, N, HV), jnp.float32),
            jax.ShapeDtypeStruct((NC, N, LANES), jnp.float32),
        ],
        mesh=mesh,
        compiler_params=pltpu.CompilerParams(use_tc_tiling_on_sc=False),
        scratch_types=[
            pltpu.VMEM((C,), jnp.int32),
            pltpu.VMEM((C,), jnp.int32),
            pltpu.VMEM((C,), jnp.int32),
            pltpu.VMEM((C, W), jnp.float32),
            pltpu.VMEM((C, W), jnp.float32),
            pltpu.VMEM((C, HV), jnp.float32),
            pltpu.VMEM((C, LANES), jnp.float32),
            pltpu.VMEM_SHARED((N, HV), jnp.float32),
            pltpu.VMEM_SHARED((N, LANES), jnp.float32),
            pltpu.SemaphoreType.DMA,
            pltpu.SemaphoreType.DMA,
        ],
    )
    return f(A2, B2, src, dst)


def _finalize_body(dv_ref, dsp_ref, dvf_ref, ds_ref):
    dvf_ref[...] = jnp.concatenate(
        [dv_ref[c, p] for c in range(NC) for p in range(NP)], axis=1)
    ds_ref[...] = jnp.sum(dsp_ref[...], axis=(0, 2))[None, :]


def _finalize(dv, dsp, N, HV, bn=2000):
    grid = (N // bn,)
    return pl.pallas_call(
        _finalize_body,
        grid=grid,
        in_specs=[
            pl.BlockSpec((NC, NP, bn, HV), lambda i: (0, 0, i, 0)),
            pl.BlockSpec((NC, N, LANES), lambda i: (0, 0, 0)),
        ],
        out_specs=[
            pl.BlockSpec((bn, NC * NP * HV), lambda i: (i, 0)),
            pl.BlockSpec((1, N), lambda i: (0, 0)),
        ],
        out_shape=[
            jax.ShapeDtypeStruct((N, NC * NP * HV), jnp.float32),
            jax.ShapeDtypeStruct((1, N), jnp.float32),
        ],
    )(dv, dsp)


def kernel(s, v, edge_index, r_ij, W1, b1, W2, b2, W3, b3, W4, b4, W5, b5):
    N, F = s.shape
    E = edge_index.shape[1]
    HV = 3 * (F // (NC * NP))
    W = 7 * (F // (NC * NP))

    v384 = v.reshape(N, 3 * F)
    r_ij_T = r_ij.T
    src = edge_index[0].astype(jnp.int32)
    dst = edge_index[1].astype(jnp.int32)

    R3 = jnp.repeat(jnp.eye(F, dtype=jnp.float32), 3, axis=1)
    T3 = jnp.tile(jnp.eye(3, dtype=jnp.float32), (1, F))
    Wcat = jnp.concatenate([W2, W3, W1], axis=1)
    bcat = jnp.concatenate([b2, b3, b1])[None, :]
    one11 = jnp.ones((1, 1), jnp.float32)

    A = _node_tables(s, v384, W4, b4[None, :], W5[:, :2 * F],
                     b5[None, :2 * F], R3)
    B = _edge_tables(r_ij_T, Wcat, bcat, T3, one11, F)

    A2 = A.reshape(NC * NP * N, W)
    B2 = B.reshape(NC * NP * E, W)

    dv, dsp = _sc_message(A2, B2, src, dst, N, E, HV, W)

    dvf, ds2 = _finalize(dv, dsp, N, HV)
    delta_s = ds2.reshape(N)
    delta_v = dvf.reshape(N, F, 3)
    return (delta_s, delta_v)
